# Initial kernel scaffold; baseline (speedup 1.0000x reference)
import jax
import jax.numpy as jnp
from jax.experimental import pallas as pl


def kernel(x, PE_weight):
    raise NotImplementedError("write your pallas kernel here")


# speedup vs baseline: 2.1822x; 2.1822x over previous
import functools

import jax
import jax.numpy as jnp
from jax import lax
from jax.experimental import pallas as pl
from jax.experimental.pallas import tpu as pltpu
from jax.experimental.pallas import tpu_sc as plsc

_BATCH = 4
_SEQ = 8192
_VOCAB = 8192
_D = 1024
_N = _BATCH * _SEQ
_NC = 2
_NS = 16
_NW = _NC * _NS
_PER_W = _N // _NW
_C = 64
_NCHUNK = _PER_W // _C

_mesh = plsc.VectorSubcoreMesh(core_axis_name="c", subcore_axis_name="s")


@functools.partial(
    pl.kernel,
    mesh=_mesh,
    out_type=jax.ShapeDtypeStruct((_N, _D), jnp.float32),
    scratch_types=[
        pltpu.VMEM((_NCHUNK, _C), jnp.int32),
        pltpu.VMEM((_C, _D), jnp.float32),
        pltpu.SemaphoreType.DMA,
    ],
)
def _embed_lookup(table_hbm, idx_hbm, out_hbm, idx_v, rows_v, gsem):
    wid = lax.axis_index("s") * _NC + lax.axis_index("c")
    base = wid * _PER_W
    pltpu.sync_copy(idx_hbm.at[wid], idx_v)

    def body(j, carry):
        pltpu.async_copy(table_hbm.at[idx_v.at[j]], rows_v, gsem).wait()
        pltpu.sync_copy(rows_v, out_hbm.at[pl.ds(base + j * _C, _C)])
        return carry

    lax.fori_loop(0, _NCHUNK, body, 0)


def kernel(x, PE_weight):
    idx = x.reshape(_NW, _NCHUNK, _C).astype(jnp.int32)
    out = _embed_lookup(PE_weight, idx)
    return out.reshape(_BATCH, _SEQ, _D)


# speedup vs baseline: 2.3664x; 1.0844x over previous
import functools

import jax
import jax.numpy as jnp
from jax import lax
from jax.experimental import pallas as pl
from jax.experimental.pallas import tpu as pltpu
from jax.experimental.pallas import tpu_sc as plsc

_BATCH = 4
_SEQ = 8192
_VOCAB = 8192
_D = 1024
_N = _BATCH * _SEQ
_NC = 2
_NS = 16
_NW = _NC * _NS
_PER_W = _N // _NW
_C = 32
_NCHUNK = _PER_W // _C
_NBUF = 2

_mesh = plsc.VectorSubcoreMesh(core_axis_name="c", subcore_axis_name="s")


@functools.partial(
    pl.kernel,
    mesh=_mesh,
    out_type=jax.ShapeDtypeStruct((_N, _D), jnp.float32),
    scratch_types=[
        pltpu.VMEM((_NCHUNK, _C), jnp.int32),
        pltpu.VMEM((_NBUF, _C, _D), jnp.float32),
    ]
    + [pltpu.SemaphoreType.DMA] * (2 * _NBUF),
)
def _embed_lookup(table_hbm, idx_hbm, out_hbm, idx_v, rows_v, *sems):
    gsems = sems[:_NBUF]
    ssems = sems[_NBUF:]
    wid = lax.axis_index("s") * _NC + lax.axis_index("c")
    base = wid * _PER_W
    pltpu.sync_copy(idx_hbm.at[wid], idx_v)

    def gather_start(j, b):
        pltpu.make_async_copy(
            table_hbm.at[idx_v.at[j]], rows_v.at[b], gsems[b]
        ).start()

    def gather_wait(j, b):
        pltpu.make_async_copy(
            table_hbm.at[idx_v.at[j]], rows_v.at[b], gsems[b]
        ).wait()

    def store_start(j, b):
        pltpu.make_async_copy(
            rows_v.at[b], out_hbm.at[pl.ds(base + j * _C, _C)], ssems[b]
        ).start()

    def store_wait(j, b):
        pltpu.make_async_copy(
            rows_v.at[b], out_hbm.at[pl.ds(base + j * _C, _C)], ssems[b]
        ).wait()

    for b in range(_NBUF):
        gather_start(b, b)

    def body(g, carry):
        for b in range(_NBUF):
            j = g * _NBUF + b
            gather_wait(j, b)
            store_start(j, b)
            nxt = j + _NBUF

            @pl.when(nxt < _NCHUNK)
            def _():
                store_wait(j, b)
                gather_start(nxt, b)

        return carry

    lax.fori_loop(0, _NCHUNK // _NBUF, body, 0)

    for b in range(_NBUF):
        store_wait(_NCHUNK - _NBUF + b, b)


def kernel(x, PE_weight):
    idx = x.reshape(_NW, _NCHUNK, _C).astype(jnp.int32)
    out = _embed_lookup(PE_weight, idx)
    return out.reshape(_BATCH, _SEQ, _D)


# speedup vs baseline: 2.3708x; 1.0019x over previous
import functools

import jax
import jax.numpy as jnp
from jax import lax
from jax.experimental import pallas as pl
from jax.experimental.pallas import tpu as pltpu
from jax.experimental.pallas import tpu_sc as plsc

_BATCH = 4
_SEQ = 8192
_VOCAB = 8192
_D = 1024
_N = _BATCH * _SEQ
_NC = 2
_NS = 16
_NW = _NC * _NS
_PER_W = _N // _NW
_C = 16
_NCHUNK = _PER_W // _C
_NBUF = 4

_mesh = plsc.VectorSubcoreMesh(core_axis_name="c", subcore_axis_name="s")


@functools.partial(
    pl.kernel,
    mesh=_mesh,
    out_type=jax.ShapeDtypeStruct((_N, _D), jnp.float32),
    scratch_types=[
        pltpu.VMEM((_NCHUNK, _C), jnp.int32),
        pltpu.VMEM((_NBUF, _C, _D), jnp.float32),
    ]
    + [pltpu.SemaphoreType.DMA] * (2 * _NBUF),
)
def _embed_lookup(table_hbm, idx_hbm, out_hbm, idx_v, rows_v, *sems):
    gsems = sems[:_NBUF]
    ssems = sems[_NBUF:]
    wid = lax.axis_index("s") * _NC + lax.axis_index("c")
    base = wid * _PER_W
    pltpu.sync_copy(idx_hbm.at[wid], idx_v)

    def gather_start(j, b):
        pltpu.make_async_copy(
            table_hbm.at[idx_v.at[j]], rows_v.at[b], gsems[b]
        ).start()

    def gather_wait(j, b):
        pltpu.make_async_copy(
            table_hbm.at[idx_v.at[j]], rows_v.at[b], gsems[b]
        ).wait()

    def store_start(j, b):
        pltpu.make_async_copy(
            rows_v.at[b], out_hbm.at[pl.ds(base + j * _C, _C)], ssems[b]
        ).start()

    def store_wait(j, b):
        pltpu.make_async_copy(
            rows_v.at[b], out_hbm.at[pl.ds(base + j * _C, _C)], ssems[b]
        ).wait()

    for b in range(_NBUF):
        gather_start(b, b)

    def body(g, carry):
        for b in range(_NBUF):
            j = g * _NBUF + b
            gather_wait(j, b)
            store_start(j, b)
            nxt = j + _NBUF

            @pl.when(nxt < _NCHUNK)
            def _():
                store_wait(j, b)
                gather_start(nxt, b)

        return carry

    lax.fori_loop(0, _NCHUNK // _NBUF, body, 0)

    for b in range(_NBUF):
        store_wait(_NCHUNK - _NBUF + b, b)


def kernel(x, PE_weight):
    idx = x.reshape(_NW, _NCHUNK, _C).astype(jnp.int32)
    out = _embed_lookup(PE_weight, idx)
    return out.reshape(_BATCH, _SEQ, _D)
